# baseline (speedup 1.0000x reference)
import functools

import jax
import jax.numpy as jnp
from jax import lax
from jax.experimental import pallas as pl
from jax.experimental.pallas import tpu as pltpu
from jax.experimental.pallas import tpu_sc as plsc

NUM_CHANNELS = 100000
EMBED_DIM = 64
BATCH = 16384
DATE_DIM = 5

NC = 2
NS = 16
L = 16
NW = NC * NS
BPW = BATCH // NW
CHUNK = 128
NCHUNK = BPW // CHUNK
DVEC = EMBED_DIM // L

_mesh = plsc.VectorSubcoreMesh(core_axis_name="c", subcore_axis_name="s")


@functools.partial(
    pl.kernel,
    mesh=_mesh,
    out_type=jax.ShapeDtypeStruct((BATCH, 2 * EMBED_DIM), jnp.float32),
    scratch_types=[
        pltpu.VMEM((NCHUNK, CHUNK), jnp.int32),
        pltpu.VMEM((BPW, EMBED_DIM), jnp.float32),
        pltpu.VMEM((BPW, 2 * EMBED_DIM), jnp.float32),
        pltpu.VMEM((BPW * DATE_DIM + L,), jnp.float32),
        pltpu.VMEM((DATE_DIM, EMBED_DIM), jnp.float32),
        pltpu.VMEM((EMBED_DIM,), jnp.float32),
        pltpu.SemaphoreType.DMA,
    ],
    compiler_params=pltpu.CompilerParams(use_tc_tiling_on_sc=False),
)
def _metadata_branch(date_hbm, idx_hbm, table_hbm, wt_hbm, bias_hbm, out_hbm,
                     idx_v, rows_v, comb_v, date_v, wt_v, bias_v, gsem):
    wid = lax.axis_index("s") * NC + lax.axis_index("c")
    base = wid * BPW

    pltpu.sync_copy(idx_hbm.at[wid], idx_v)
    pltpu.sync_copy(date_hbm.at[wid], date_v.at[pl.ds(0, BPW * DATE_DIM)])
    pltpu.sync_copy(wt_hbm, wt_v)
    pltpu.sync_copy(bias_hbm, bias_v)

    copies = []
    for j in range(NCHUNK):
        copies.append(
            pltpu.async_copy(
                table_hbm.at[idx_v.at[j]],
                rows_v.at[pl.ds(j * CHUNK, CHUNK)],
                gsem,
            )
        )

    wvec = [[wt_v[k, pl.ds(d * L, L)] for d in range(DVEC)]
            for k in range(DATE_DIM)]
    bvec = [bias_v[pl.ds(d * L, L)] for d in range(DVEC)]

    def row_body(b, carry):
        svec = date_v[pl.ds(b * DATE_DIM, L)]
        s = [svec[k] for k in range(DATE_DIM)]
        for d in range(DVEC):
            acc = bvec[d]
            for k in range(DATE_DIM):
                acc = acc + s[k] * wvec[k][d]
            comb_v[b, pl.ds(d * L, L)] = acc
        return carry

    lax.fori_loop(0, BPW, row_body, 0)

    for c in copies:
        c.wait()

    def copy_body(b, carry):
        for d in range(DVEC):
            comb_v[b, pl.ds(EMBED_DIM + d * L, L)] = rows_v[b, pl.ds(d * L, L)]
        return carry

    lax.fori_loop(0, BPW, copy_body, 0)
    pltpu.sync_copy(comb_v, out_hbm.at[pl.ds(base, BPW)])


def kernel(date_features, channel_ids, channel_table, date_W, date_b):
    idx = channel_ids.astype(jnp.int32).reshape(NW, NCHUNK, CHUNK)
    date = date_features.reshape(NW, BPW * DATE_DIM)
    wt = date_W.T
    return _metadata_branch(date, idx, channel_table, wt, date_b)


# speedup vs baseline: 1.1214x; 1.1214x over previous
import functools

import jax
import jax.numpy as jnp
from jax import lax
from jax.experimental import pallas as pl
from jax.experimental.pallas import tpu as pltpu
from jax.experimental.pallas import tpu_sc as plsc

NUM_CHANNELS = 100000
EMBED_DIM = 64
BATCH = 16384
DATE_DIM = 5

NC = 2
NS = 16
L = 16
NW = NC * NS
BPW = BATCH // NW
CHUNK = 128
NCHUNK = BPW // CHUNK
DVEC = EMBED_DIM // L

_mesh = plsc.VectorSubcoreMesh(core_axis_name="c", subcore_axis_name="s")


@functools.partial(
    pl.kernel,
    mesh=_mesh,
    out_type=jax.ShapeDtypeStruct((BATCH, 2 * EMBED_DIM), jnp.float32),
    scratch_types=[
        pltpu.VMEM((BPW,), jnp.int32),
        pltpu.VMEM((BPW, EMBED_DIM), jnp.float32),
        pltpu.VMEM((BPW, EMBED_DIM), jnp.float32),
        pltpu.VMEM((DATE_DIM, BPW), jnp.float32),
        pltpu.VMEM((EMBED_DIM, DATE_DIM), jnp.float32),
        pltpu.VMEM((EMBED_DIM,), jnp.float32),
        pltpu.SemaphoreType.DMA,
    ],
    compiler_params=pltpu.CompilerParams(use_tc_tiling_on_sc=False,
                                         needs_layout_passes=False),
)
def _metadata_branch(date_hbm, idx_hbm, table_hbm, w_hbm, bias_hbm, out_hbm,
                     idx_v, rows_v, demb_v, date_v, w_v, bias_v, gsem):
    wid = lax.axis_index("s") * NC + lax.axis_index("c")
    base = wid * BPW

    pltpu.sync_copy(idx_hbm.at[pl.ds(base, BPW)], idx_v)
    copies = []
    for j in range(NCHUNK):
        copies.append(
            pltpu.async_copy(
                table_hbm.at[idx_v.at[pl.ds(j * CHUNK, CHUNK)]],
                rows_v.at[pl.ds(j * CHUNK, CHUNK)],
                gsem,
            )
        )

    pltpu.sync_copy(date_hbm.at[:, pl.ds(base, BPW)], date_v)
    pltpu.sync_copy(w_hbm, w_v)
    pltpu.sync_copy(bias_hbm, bias_v)

    lanes = lax.iota(jnp.int32, L)
    wvec = [
        [plsc.load_gather(w_v, [lanes + d * L, jnp.full((L,), k, jnp.int32)])
         for d in range(DVEC)]
        for k in range(DATE_DIM)
    ]
    bvec = [bias_v[pl.ds(d * L, L)] for d in range(DVEC)]

    def row_body(b, carry):
        s = [
            plsc.load_gather(
                date_v,
                [jnp.full((L,), k, jnp.int32),
                 jnp.full((L,), b, jnp.int32)],
            )
            for k in range(DATE_DIM)
        ]
        for d in range(DVEC):
            acc = bvec[d]
            for k in range(DATE_DIM):
                acc = acc + s[k] * wvec[k][d]
            demb_v[b, pl.ds(d * L, L)] = acc
        return carry

    lax.fori_loop(0, BPW, row_body, 0, unroll=4)

    pltpu.sync_copy(demb_v, out_hbm.at[pl.ds(base, BPW), pl.ds(0, EMBED_DIM)])
    for c in copies:
        c.wait()
    pltpu.sync_copy(rows_v,
                    out_hbm.at[pl.ds(base, BPW), pl.ds(EMBED_DIM, EMBED_DIM)])


def kernel(date_features, channel_ids, channel_table, date_W, date_b):
    return _metadata_branch(date_features.T, channel_ids.astype(jnp.int32),
                            channel_table, date_W, date_b)


# speedup vs baseline: 1.2345x; 1.1008x over previous
import functools

import jax
import jax.numpy as jnp
from jax import lax
from jax.experimental import pallas as pl
from jax.experimental.pallas import tpu as pltpu
from jax.experimental.pallas import tpu_sc as plsc

NUM_CHANNELS = 100000
EMBED_DIM = 64
BATCH = 16384
DATE_DIM = 5

NC = 2
NS = 16
NW = NC * NS
BPW = BATCH // NW
CHUNK = 128
NCHUNK = BPW // CHUNK

RB = 2048

_mesh = plsc.VectorSubcoreMesh(core_axis_name="c", subcore_axis_name="s")


def _date_body(dt_ref, w_ref, b_ref, out_ref):
    de = lax.dot_general(dt_ref[...], w_ref[...], (((0,), (1,)), ((), ())),
                         preferred_element_type=jnp.float32)
    out_ref[:, 0:EMBED_DIM] = de + b_ref[...]


_date_proj = pl.pallas_call(
    _date_body,
    out_shape=jax.ShapeDtypeStruct((BATCH, 2 * EMBED_DIM), jnp.float32),
    grid=(BATCH // RB,),
    in_specs=[
        pl.BlockSpec((DATE_DIM, RB), lambda i: (0, i)),
        pl.BlockSpec((EMBED_DIM, DATE_DIM), lambda i: (0, 0)),
        pl.BlockSpec((1, EMBED_DIM), lambda i: (0, 0)),
    ],
    out_specs=pl.BlockSpec((RB, 2 * EMBED_DIM), lambda i: (i, 0)),
)


@functools.partial(
    pl.kernel,
    mesh=_mesh,
    out_type=(),
    scratch_types=[
        pltpu.VMEM((BPW,), jnp.int32),
        pltpu.VMEM((BPW, EMBED_DIM), jnp.float32),
        pltpu.SemaphoreType.DMA,
    ],
    compiler_params=pltpu.CompilerParams(use_tc_tiling_on_sc=False),
)
def _sc_gather(idx_hbm, table_hbm, out_hbm, idx_v, rows_v, gsem):
    wid = lax.axis_index("s") * NC + lax.axis_index("c")
    base = wid * BPW

    pltpu.sync_copy(idx_hbm.at[pl.ds(base, BPW)], idx_v)
    copies = []
    for j in range(NCHUNK):
        copies.append(
            pltpu.async_copy(
                table_hbm.at[idx_v.at[pl.ds(j * CHUNK, CHUNK)]],
                rows_v.at[pl.ds(j * CHUNK, CHUNK)],
                gsem,
            )
        )
    for c in copies:
        c.wait()
    pltpu.sync_copy(rows_v,
                    out_hbm.at[pl.ds(base, BPW), pl.ds(EMBED_DIM, EMBED_DIM)])


def kernel(date_features, channel_ids, channel_table, date_W, date_b):
    out0 = _date_proj(date_features.T, date_W, date_b.reshape(1, EMBED_DIM))
    out_ref = jax.new_ref(out0)
    _sc_gather(channel_ids.astype(jnp.int32), channel_table, out_ref)
    return out_ref[...]
